# baseline (speedup 1.0000x reference)
import jax
import jax.numpy as jnp
from jax.experimental import pallas as pl
from jax.experimental.pallas import tpu as pltpu

_CH = 1000
_NBUF = 5


def _gates_body(zl_hbm, zh_hbm, w1t_ref, b1_ref, w2t_ref, b2_ref,
                ol_hbm, oh_hbm, bufl, bufh, ostl, osth, sems, osems):
    n = zl_hbm.shape[0]
    nch = n // _CH

    def start(i, slot):
        pltpu.make_async_copy(
            zl_hbm.at[pl.ds(i * _CH, _CH), :], bufl.at[slot], sems.at[0, slot]
        ).start()
        pltpu.make_async_copy(
            zh_hbm.at[pl.ds(i * _CH, _CH), :], bufh.at[slot], sems.at[1, slot]
        ).start()

    for s in range(_NBUF - 1):
        start(s, s)

    def body(i, carry):
        slot = jax.lax.rem(i, _NBUF)
        nxt = i + (_NBUF - 1)

        @pl.when(nxt < nch)
        def _():
            start(nxt, jax.lax.rem(nxt, _NBUF))

        pltpu.make_async_copy(
            zl_hbm.at[pl.ds(i * _CH, _CH), :], bufl.at[slot], sems.at[0, slot]
        ).wait()
        pltpu.make_async_copy(
            zh_hbm.at[pl.ds(i * _CH, _CH), :], bufh.at[slot], sems.at[1, slot]
        ).wait()

        @pl.when(i >= _NBUF)
        def _():
            pltpu.make_async_copy(
                ostl.at[slot], ol_hbm.at[pl.ds(0, _CH), :], osems.at[0, slot]
            ).wait()
            pltpu.make_async_copy(
                osth.at[slot], oh_hbm.at[pl.ds(0, _CH), :], osems.at[1, slot]
            ).wait()

        ostl[slot] = (
            jnp.dot(bufl[slot], w1t_ref[...], preferred_element_type=jnp.float32)
            + b1_ref[...]
        )
        osth[slot] = (
            jnp.dot(bufh[slot], w2t_ref[...], preferred_element_type=jnp.float32)
            + b2_ref[...]
        )
        pltpu.make_async_copy(
            ostl.at[slot], ol_hbm.at[pl.ds(i * _CH, _CH), :], osems.at[0, slot]
        ).start()
        pltpu.make_async_copy(
            osth.at[slot], oh_hbm.at[pl.ds(i * _CH, _CH), :], osems.at[1, slot]
        ).start()
        return carry

    jax.lax.fori_loop(0, nch, body, 0)

    def drain(j, carry):
        i = nch - 1 - j
        slot = jax.lax.rem(i, _NBUF)
        pltpu.make_async_copy(
            ostl.at[slot], ol_hbm.at[pl.ds(i * _CH, _CH), :], osems.at[0, slot]
        ).wait()
        pltpu.make_async_copy(
            osth.at[slot], oh_hbm.at[pl.ds(i * _CH, _CH), :], osems.at[1, slot]
        ).wait()
        return carry

    jax.lax.fori_loop(0, min(_NBUF, nch), drain, 0)


@jax.jit
def kernel(z_lp, z_hp, W1, b1, W2, b2):
    n, d = z_lp.shape
    w1t = W1.T
    w2t = W2.T
    b1r = b1.reshape(1, 2)
    b2r = b2.reshape(1, 2)
    out_shape = (
        jax.ShapeDtypeStruct((n, 2), jnp.float32),
        jax.ShapeDtypeStruct((n, 2), jnp.float32),
    )
    x_lp, x_hp = pl.pallas_call(
        _gates_body,
        in_specs=[
            pl.BlockSpec(memory_space=pltpu.MemorySpace.HBM),
            pl.BlockSpec(memory_space=pltpu.MemorySpace.HBM),
            pl.BlockSpec(memory_space=pltpu.MemorySpace.VMEM),
            pl.BlockSpec(memory_space=pltpu.MemorySpace.VMEM),
            pl.BlockSpec(memory_space=pltpu.MemorySpace.VMEM),
            pl.BlockSpec(memory_space=pltpu.MemorySpace.VMEM),
        ],
        out_specs=(
            pl.BlockSpec(memory_space=pltpu.MemorySpace.HBM),
            pl.BlockSpec(memory_space=pltpu.MemorySpace.HBM),
        ),
        out_shape=out_shape,
        scratch_shapes=[
            pltpu.VMEM((_NBUF, _CH, d), jnp.float32),
            pltpu.VMEM((_NBUF, _CH, d), jnp.float32),
            pltpu.VMEM((_NBUF, _CH, 2), jnp.float32),
            pltpu.VMEM((_NBUF, _CH, 2), jnp.float32),
            pltpu.SemaphoreType.DMA((2, _NBUF)),
            pltpu.SemaphoreType.DMA((2, _NBUF)),
        ],
    )(z_lp, z_hp, w1t, b1r, w2t, b2r)
    return (x_lp, x_hp)


# speedup vs baseline: 1.2122x; 1.2122x over previous
import functools

import jax
import jax.numpy as jnp
from jax import lax
from jax.experimental import pallas as pl
from jax.experimental.pallas import tpu as pltpu
from jax.experimental.pallas import tpu_sc as plsc

_N = 100000
_D = 1024

_NS = 25600
_NW = 32
_RPW = _NS // _NW
_R = 16
_SNB = 2
_NCHUNK = _RPW // _R
_GROUPS = _NCHUNK // _SNB

_CH = 744
_NBUF = 5


def _tc_body(zl_hbm, zh_hbm, w1t_ref, b1_ref, w2t_ref, b2_ref,
             ol_ref, oh_ref, bufl, bufh, sems):
    nch = (_N - _NS) // _CH

    def start(i, slot):
        pltpu.make_async_copy(
            zl_hbm.at[pl.ds(_NS + i * _CH, _CH), :], bufl.at[slot],
            sems.at[0, slot]).start()
        pltpu.make_async_copy(
            zh_hbm.at[pl.ds(_NS + i * _CH, _CH), :], bufh.at[slot],
            sems.at[1, slot]).start()

    for s in range(_NBUF - 1):
        start(s, s)

    def body(i, carry):
        slot = jax.lax.rem(i, _NBUF)
        nxt = i + (_NBUF - 1)

        @pl.when(nxt < nch)
        def _():
            start(nxt, jax.lax.rem(nxt, _NBUF))

        pltpu.make_async_copy(
            zl_hbm.at[pl.ds(_NS + i * _CH, _CH), :], bufl.at[slot],
            sems.at[0, slot]).wait()
        pltpu.make_async_copy(
            zh_hbm.at[pl.ds(_NS + i * _CH, _CH), :], bufh.at[slot],
            sems.at[1, slot]).wait()

        ol_ref[i] = (
            lax.dot_general(w1t_ref[...], bufl[slot], (((0,), (1,)), ((), ())),
                            preferred_element_type=jnp.float32)
            + b1_ref[...]
        )
        oh_ref[i] = (
            lax.dot_general(w2t_ref[...], bufh[slot], (((0,), (1,)), ((), ())),
                            preferred_element_type=jnp.float32)
            + b2_ref[...]
        )
        return carry

    jax.lax.fori_loop(0, nch, body, 0)


def _tc_call(z_lp, z_hp, W1, b1, W2, b2):
    nch = (_N - _NS) // _CH
    w1t = W1.T
    w2t = W2.T
    b1r = b1.reshape(2, 1)
    b2r = b2.reshape(2, 1)
    out_shape = (
        jax.ShapeDtypeStruct((nch, 2, _CH), jnp.float32),
        jax.ShapeDtypeStruct((nch, 2, _CH), jnp.float32),
    )
    return pl.pallas_call(
        _tc_body,
        in_specs=[
            pl.BlockSpec(memory_space=pltpu.MemorySpace.HBM),
            pl.BlockSpec(memory_space=pltpu.MemorySpace.HBM),
            pl.BlockSpec(memory_space=pltpu.MemorySpace.VMEM),
            pl.BlockSpec(memory_space=pltpu.MemorySpace.VMEM),
            pl.BlockSpec(memory_space=pltpu.MemorySpace.VMEM),
            pl.BlockSpec(memory_space=pltpu.MemorySpace.VMEM),
        ],
        out_specs=(
            pl.BlockSpec(memory_space=pltpu.MemorySpace.VMEM),
            pl.BlockSpec(memory_space=pltpu.MemorySpace.VMEM),
        ),
        out_shape=out_shape,
        scratch_shapes=[
            pltpu.VMEM((_NBUF, _CH, _D), jnp.float32),
            pltpu.VMEM((_NBUF, _CH, _D), jnp.float32),
            pltpu.SemaphoreType.DMA((2, _NBUF)),
        ],
    )(z_lp, z_hp, w1t, b1r, w2t, b2r)


_GD = lax.GatherDimensionNumbers(
    offset_dims=(), collapsed_slice_dims=(0,), start_index_map=(0,))


def _allsum(v, rows16):
    for sh in (8, 4, 2, 1):
        idx = ((rows16 + sh) & 15).reshape(16, 1)
        v = v + lax.gather(v, idx, _GD, (1,),
                           mode=lax.GatherScatterMode.PROMISE_IN_BOUNDS)
    return v


def _sc_chunk(buf, w_v, rows16, zeros):
    def kbody(kc, accs):
        w0 = w_v[0, pl.ds(kc * 16, 16)]
        w1 = w_v[1, pl.ds(kc * 16, 16)]
        new = []
        for rr in range(16):
            z = buf[rr, pl.ds(kc * 16, 16)]
            new.append(accs[2 * rr] + z * w0)
            new.append(accs[2 * rr + 1] + z * w1)
        return tuple(new)

    accs = jax.lax.fori_loop(0, _D // 16, kbody,
                             tuple(zeros for _ in range(32)))
    r0 = zeros
    r1 = zeros
    for rr in range(16):
        lane = rows16 == rr
        r0 = jnp.where(lane, _allsum(accs[2 * rr], rows16), r0)
        r1 = jnp.where(lane, _allsum(accs[2 * rr + 1], rows16), r1)
    return r0, r1


def _sc_kernel_body(zl_hbm, zh_hbm, w1_hbm, w2_hbm, outl_hbm, outh_hbm,
                    bufl, bufh, w1v, w2v, outlv, outhv, sems, osem):
    wid = lax.axis_index("s") * 2 + lax.axis_index("c")
    base = wid * _RPW

    pltpu.sync_copy(w1_hbm, w1v)
    pltpu.sync_copy(w2_hbm, w2v)

    rows16 = lax.iota(jnp.int32, 16)
    zeros = jnp.zeros((16,), jnp.float32)

    def start(c, b):
        pltpu.make_async_copy(
            zl_hbm.at[pl.ds(base + c * _R, _R), :], bufl.at[b],
            sems.at[0, b]).start()
        pltpu.make_async_copy(
            zh_hbm.at[pl.ds(base + c * _R, _R), :], bufh.at[b],
            sems.at[1, b]).start()

    for b in range(_SNB):
        start(b, b)

    def group(g, carry):
        for b in range(_SNB):
            c = g * _SNB + b
            pltpu.make_async_copy(
                zl_hbm.at[pl.ds(base + c * _R, _R), :], bufl.at[b],
                sems.at[0, b]).wait()
            pltpu.make_async_copy(
                zh_hbm.at[pl.ds(base + c * _R, _R), :], bufh.at[b],
                sems.at[1, b]).wait()

            al0, al1 = _sc_chunk(bufl.at[b], w1v, rows16, zeros)
            ah0, ah1 = _sc_chunk(bufh.at[b], w2v, rows16, zeros)

            nxt = c + _SNB

            @pl.when(nxt < _NCHUNK)
            def _():
                start(nxt, b)

            outlv[0, pl.ds(c * _R, _R)] = al0
            outlv[1, pl.ds(c * _R, _R)] = al1
            outhv[0, pl.ds(c * _R, _R)] = ah0
            outhv[1, pl.ds(c * _R, _R)] = ah1
        return carry

    jax.lax.fori_loop(0, _GROUPS, group, 0)

    pltpu.make_async_copy(outlv, outl_hbm.at[wid], osem.at[0]).start()
    pltpu.make_async_copy(outhv, outh_hbm.at[wid], osem.at[1]).start()
    pltpu.make_async_copy(outlv, outl_hbm.at[wid], osem.at[0]).wait()
    pltpu.make_async_copy(outhv, outh_hbm.at[wid], osem.at[1]).wait()


_sc_mesh = plsc.VectorSubcoreMesh(core_axis_name="c", subcore_axis_name="s")

_sc_gates = pl.kernel(
    _sc_kernel_body,
    mesh=_sc_mesh,
    out_type=(
        jax.ShapeDtypeStruct((_NW, 2, _RPW), jnp.float32),
        jax.ShapeDtypeStruct((_NW, 2, _RPW), jnp.float32),
    ),
    scratch_types=[
        pltpu.VMEM((_SNB, _R, _D), jnp.float32),
        pltpu.VMEM((_SNB, _R, _D), jnp.float32),
        pltpu.VMEM((2, _D), jnp.float32),
        pltpu.VMEM((2, _D), jnp.float32),
        pltpu.VMEM((2, _RPW), jnp.float32),
        pltpu.VMEM((2, _RPW), jnp.float32),
        pltpu.SemaphoreType.DMA((2, _SNB)),
        pltpu.SemaphoreType.DMA((2,)),
    ],
)


@jax.jit
def kernel(z_lp, z_hp, W1, b1, W2, b2):
    scl_t, sch_t = _sc_gates(z_lp, z_hp, W1, W2)
    tcl, tch = _tc_call(z_lp, z_hp, W1, b1, W2, b2)
    x_lp = jnp.concatenate([
        scl_t.transpose(0, 2, 1).reshape(_NS, 2) + b1.reshape(1, 2),
        tcl.transpose(0, 2, 1).reshape(_N - _NS, 2),
    ], axis=0)
    x_hp = jnp.concatenate([
        sch_t.transpose(0, 2, 1).reshape(_NS, 2) + b2.reshape(1, 2),
        tch.transpose(0, 2, 1).reshape(_N - _NS, 2),
    ], axis=0)
    return (x_lp, x_hp)


# speedup vs baseline: 1.2138x; 1.0013x over previous
import functools

import jax
import jax.numpy as jnp
from jax import lax
from jax.experimental import pallas as pl
from jax.experimental.pallas import tpu as pltpu
from jax.experimental.pallas import tpu_sc as plsc

_N = 100000
_D = 1024

_NS = 17408
_NW = 32
_RPW = _NS // _NW
_R = 16
_SNB = 2
_NCHUNK = _RPW // _R
_GROUPS = _NCHUNK // _SNB

_CH = 712
_NBUF = 5


def _tc_body(zl_hbm, zh_hbm, w1t_ref, b1_ref, w2t_ref, b2_ref,
             ol_ref, oh_ref, bufl, bufh, sems):
    nch = (_N - _NS) // _CH

    def start(i, slot):
        pltpu.make_async_copy(
            zl_hbm.at[pl.ds(_NS + i * _CH, _CH), :], bufl.at[slot],
            sems.at[0, slot]).start()
        pltpu.make_async_copy(
            zh_hbm.at[pl.ds(_NS + i * _CH, _CH), :], bufh.at[slot],
            sems.at[1, slot]).start()

    for s in range(_NBUF - 1):
        start(s, s)

    def body(i, carry):
        slot = jax.lax.rem(i, _NBUF)
        nxt = i + (_NBUF - 1)

        @pl.when(nxt < nch)
        def _():
            start(nxt, jax.lax.rem(nxt, _NBUF))

        pltpu.make_async_copy(
            zl_hbm.at[pl.ds(_NS + i * _CH, _CH), :], bufl.at[slot],
            sems.at[0, slot]).wait()
        pltpu.make_async_copy(
            zh_hbm.at[pl.ds(_NS + i * _CH, _CH), :], bufh.at[slot],
            sems.at[1, slot]).wait()

        ol_ref[i] = (
            lax.dot_general(w1t_ref[...], bufl[slot], (((0,), (1,)), ((), ())),
                            preferred_element_type=jnp.float32)
            + b1_ref[...]
        )
        oh_ref[i] = (
            lax.dot_general(w2t_ref[...], bufh[slot], (((0,), (1,)), ((), ())),
                            preferred_element_type=jnp.float32)
            + b2_ref[...]
        )
        return carry

    jax.lax.fori_loop(0, nch, body, 0)


def _tc_call(z_lp, z_hp, W1, b1, W2, b2):
    nch = (_N - _NS) // _CH
    w1t = W1.T
    w2t = W2.T
    b1r = b1.reshape(2, 1)
    b2r = b2.reshape(2, 1)
    out_shape = (
        jax.ShapeDtypeStruct((nch, 2, _CH), jnp.float32),
        jax.ShapeDtypeStruct((nch, 2, _CH), jnp.float32),
    )
    return pl.pallas_call(
        _tc_body,
        in_specs=[
            pl.BlockSpec(memory_space=pltpu.MemorySpace.HBM),
            pl.BlockSpec(memory_space=pltpu.MemorySpace.HBM),
            pl.BlockSpec(memory_space=pltpu.MemorySpace.VMEM),
            pl.BlockSpec(memory_space=pltpu.MemorySpace.VMEM),
            pl.BlockSpec(memory_space=pltpu.MemorySpace.VMEM),
            pl.BlockSpec(memory_space=pltpu.MemorySpace.VMEM),
        ],
        out_specs=(
            pl.BlockSpec(memory_space=pltpu.MemorySpace.VMEM),
            pl.BlockSpec(memory_space=pltpu.MemorySpace.VMEM),
        ),
        out_shape=out_shape,
        scratch_shapes=[
            pltpu.VMEM((_NBUF, _CH, _D), jnp.float32),
            pltpu.VMEM((_NBUF, _CH, _D), jnp.float32),
            pltpu.SemaphoreType.DMA((2, _NBUF)),
        ],
    )(z_lp, z_hp, w1t, b1r, w2t, b2r)


_GD = lax.GatherDimensionNumbers(
    offset_dims=(), collapsed_slice_dims=(0,), start_index_map=(0,))


def _allsum(v, rows16):
    for sh in (8, 4, 2, 1):
        idx = ((rows16 + sh) & 15).reshape(16, 1)
        v = v + lax.gather(v, idx, _GD, (1,),
                           mode=lax.GatherScatterMode.PROMISE_IN_BOUNDS)
    return v


def _sc_chunk(buf, w_v, rows16, zeros):
    def kbody(kc, accs):
        w0 = w_v[0, pl.ds(kc * 16, 16)]
        w1 = w_v[1, pl.ds(kc * 16, 16)]
        new = []
        for rr in range(16):
            z = buf[rr, pl.ds(kc * 16, 16)]
            new.append(accs[2 * rr] + z * w0)
            new.append(accs[2 * rr + 1] + z * w1)
        return tuple(new)

    accs = jax.lax.fori_loop(0, _D // 16, kbody,
                             tuple(zeros for _ in range(32)))
    r0 = zeros
    r1 = zeros
    for rr in range(16):
        lane = rows16 == rr
        r0 = jnp.where(lane, _allsum(accs[2 * rr], rows16), r0)
        r1 = jnp.where(lane, _allsum(accs[2 * rr + 1], rows16), r1)
    return r0, r1


def _sc_kernel_body(zl_hbm, zh_hbm, w1_hbm, w2_hbm, outl_hbm, outh_hbm,
                    bufl, bufh, w1v, w2v, outlv, outhv, sems, osem):
    wid = lax.axis_index("s") * 2 + lax.axis_index("c")
    base = wid * _RPW

    pltpu.sync_copy(w1_hbm, w1v)
    pltpu.sync_copy(w2_hbm, w2v)

    rows16 = lax.iota(jnp.int32, 16)
    zeros = jnp.zeros((16,), jnp.float32)

    def start(c, b):
        pltpu.make_async_copy(
            zl_hbm.at[pl.ds(base + c * _R, _R), :], bufl.at[b],
            sems.at[0, b]).start()
        pltpu.make_async_copy(
            zh_hbm.at[pl.ds(base + c * _R, _R), :], bufh.at[b],
            sems.at[1, b]).start()

    for b in range(_SNB):
        start(b, b)

    def group(g, carry):
        for b in range(_SNB):
            c = g * _SNB + b
            pltpu.make_async_copy(
                zl_hbm.at[pl.ds(base + c * _R, _R), :], bufl.at[b],
                sems.at[0, b]).wait()
            pltpu.make_async_copy(
                zh_hbm.at[pl.ds(base + c * _R, _R), :], bufh.at[b],
                sems.at[1, b]).wait()

            al0, al1 = _sc_chunk(bufl.at[b], w1v, rows16, zeros)
            ah0, ah1 = _sc_chunk(bufh.at[b], w2v, rows16, zeros)

            nxt = c + _SNB

            @pl.when(nxt < _NCHUNK)
            def _():
                start(nxt, b)

            outlv[0, pl.ds(c * _R, _R)] = al0
            outlv[1, pl.ds(c * _R, _R)] = al1
            outhv[0, pl.ds(c * _R, _R)] = ah0
            outhv[1, pl.ds(c * _R, _R)] = ah1
        return carry

    jax.lax.fori_loop(0, _GROUPS, group, 0)

    pltpu.make_async_copy(outlv, outl_hbm.at[wid], osem.at[0]).start()
    pltpu.make_async_copy(outhv, outh_hbm.at[wid], osem.at[1]).start()
    pltpu.make_async_copy(outlv, outl_hbm.at[wid], osem.at[0]).wait()
    pltpu.make_async_copy(outhv, outh_hbm.at[wid], osem.at[1]).wait()


_sc_mesh = plsc.VectorSubcoreMesh(core_axis_name="c", subcore_axis_name="s")

_sc_gates = pl.kernel(
    _sc_kernel_body,
    mesh=_sc_mesh,
    out_type=(
        jax.ShapeDtypeStruct((_NW, 2, _RPW), jnp.float32),
        jax.ShapeDtypeStruct((_NW, 2, _RPW), jnp.float32),
    ),
    scratch_types=[
        pltpu.VMEM((_SNB, _R, _D), jnp.float32),
        pltpu.VMEM((_SNB, _R, _D), jnp.float32),
        pltpu.VMEM((2, _D), jnp.float32),
        pltpu.VMEM((2, _D), jnp.float32),
        pltpu.VMEM((2, _RPW), jnp.float32),
        pltpu.VMEM((2, _RPW), jnp.float32),
        pltpu.SemaphoreType.DMA((2, _SNB)),
        pltpu.SemaphoreType.DMA((2,)),
    ],
)


@jax.jit
def kernel(z_lp, z_hp, W1, b1, W2, b2):
    scl_t, sch_t = _sc_gates(z_lp, z_hp, W1, W2)
    tcl, tch = _tc_call(z_lp, z_hp, W1, b1, W2, b2)
    x_lp = jnp.concatenate([
        scl_t.transpose(0, 2, 1).reshape(_NS, 2) + b1.reshape(1, 2),
        tcl.transpose(0, 2, 1).reshape(_N - _NS, 2),
    ], axis=0)
    x_hp = jnp.concatenate([
        sch_t.transpose(0, 2, 1).reshape(_NS, 2) + b2.reshape(1, 2),
        tch.transpose(0, 2, 1).reshape(_N - _NS, 2),
    ], axis=0)
    return (x_lp, x_hp)


# speedup vs baseline: 1.2929x; 1.0652x over previous
import jax
import jax.numpy as jnp
from jax import lax
from jax.experimental import pallas as pl
from jax.experimental.pallas import tpu as pltpu

_CH = 1000
_NBUF = 5

_DN = (((0,), (1,)), ((), ()))


def _gates_body(zl_hbm, zh_hbm, w1t_ref, b1_ref, w2t_ref, b2_ref,
                ol_ref, oh_ref, bufl, bufh, sems):
    n = zl_hbm.shape[0]
    nch = n // _CH

    def start(i, slot):
        pltpu.make_async_copy(
            zl_hbm.at[pl.ds(i * _CH, _CH), :], bufl.at[slot], sems.at[0, slot]
        ).start()
        pltpu.make_async_copy(
            zh_hbm.at[pl.ds(i * _CH, _CH), :], bufh.at[slot], sems.at[1, slot]
        ).start()

    for s in range(_NBUF - 1):
        start(s, s)

    def body(i, carry):
        slot = jax.lax.rem(i, _NBUF)
        nxt = i + (_NBUF - 1)

        @pl.when(nxt < nch)
        def _():
            start(nxt, jax.lax.rem(nxt, _NBUF))

        pltpu.make_async_copy(
            zl_hbm.at[pl.ds(i * _CH, _CH), :], bufl.at[slot], sems.at[0, slot]
        ).wait()
        pltpu.make_async_copy(
            zh_hbm.at[pl.ds(i * _CH, _CH), :], bufh.at[slot], sems.at[1, slot]
        ).wait()

        ol_ref[i] = (
            lax.dot_general(w1t_ref[...], bufl[slot], _DN,
                            preferred_element_type=jnp.float32)
            + b1_ref[...]
        )
        oh_ref[i] = (
            lax.dot_general(w2t_ref[...], bufh[slot], _DN,
                            preferred_element_type=jnp.float32)
            + b2_ref[...]
        )
        return carry

    jax.lax.fori_loop(0, nch, body, 0)


@jax.jit
def kernel(z_lp, z_hp, W1, b1, W2, b2):
    n, d = z_lp.shape
    w1t = W1.T
    w2t = W2.T
    b1r = b1.reshape(2, 1)
    b2r = b2.reshape(2, 1)
    nch = n // _CH
    out_shape = (
        jax.ShapeDtypeStruct((nch, 2, _CH), jnp.float32),
        jax.ShapeDtypeStruct((nch, 2, _CH), jnp.float32),
    )
    ol_t, oh_t = pl.pallas_call(
        _gates_body,
        in_specs=[
            pl.BlockSpec(memory_space=pltpu.MemorySpace.HBM),
            pl.BlockSpec(memory_space=pltpu.MemorySpace.HBM),
            pl.BlockSpec(memory_space=pltpu.MemorySpace.VMEM),
            pl.BlockSpec(memory_space=pltpu.MemorySpace.VMEM),
            pl.BlockSpec(memory_space=pltpu.MemorySpace.VMEM),
            pl.BlockSpec(memory_space=pltpu.MemorySpace.VMEM),
        ],
        out_specs=(
            pl.BlockSpec(memory_space=pltpu.MemorySpace.VMEM),
            pl.BlockSpec(memory_space=pltpu.MemorySpace.VMEM),
        ),
        out_shape=out_shape,
        scratch_shapes=[
            pltpu.VMEM((_NBUF, _CH, d), jnp.float32),
            pltpu.VMEM((_NBUF, _CH, d), jnp.float32),
            pltpu.SemaphoreType.DMA((2, _NBUF)),
        ],
    )(z_lp, z_hp, w1t, b1r, w2t, b2r)
    x_lp = ol_t.transpose(0, 2, 1).reshape(n, 2)
    x_hp = oh_t.transpose(0, 2, 1).reshape(n, 2)
    return (x_lp, x_hp)


# speedup vs baseline: 1.2981x; 1.0040x over previous
import jax
import jax.numpy as jnp
from jax import lax
from jax.experimental import pallas as pl
from jax.experimental.pallas import tpu as pltpu

_CH = 800
_NBUF = 6

_DN = (((0,), (1,)), ((), ()))


def _gates_body(zl_hbm, zh_hbm, w1t_ref, b1_ref, w2t_ref, b2_ref,
                ol_ref, oh_ref, bufl, bufh, sems):
    n = zl_hbm.shape[0]
    nch = n // _CH

    def start(i, slot):
        pltpu.make_async_copy(
            zl_hbm.at[pl.ds(i * _CH, _CH), :], bufl.at[slot], sems.at[0, slot]
        ).start()
        pltpu.make_async_copy(
            zh_hbm.at[pl.ds(i * _CH, _CH), :], bufh.at[slot], sems.at[1, slot]
        ).start()

    for s in range(_NBUF - 1):
        start(s, s)

    def body(i, carry):
        slot = jax.lax.rem(i, _NBUF)
        nxt = i + (_NBUF - 1)

        @pl.when(nxt < nch)
        def _():
            start(nxt, jax.lax.rem(nxt, _NBUF))

        pltpu.make_async_copy(
            zl_hbm.at[pl.ds(i * _CH, _CH), :], bufl.at[slot], sems.at[0, slot]
        ).wait()
        pltpu.make_async_copy(
            zh_hbm.at[pl.ds(i * _CH, _CH), :], bufh.at[slot], sems.at[1, slot]
        ).wait()

        ol_ref[i] = (
            lax.dot_general(w1t_ref[...], bufl[slot], _DN,
                            preferred_element_type=jnp.float32)
            + b1_ref[...]
        )
        oh_ref[i] = (
            lax.dot_general(w2t_ref[...], bufh[slot], _DN,
                            preferred_element_type=jnp.float32)
            + b2_ref[...]
        )
        return carry

    jax.lax.fori_loop(0, nch, body, 0)


@jax.jit
def kernel(z_lp, z_hp, W1, b1, W2, b2):
    n, d = z_lp.shape
    w1t = W1.T
    w2t = W2.T
    b1r = b1.reshape(2, 1)
    b2r = b2.reshape(2, 1)
    nch = n // _CH
    out_shape = (
        jax.ShapeDtypeStruct((nch, 2, _CH), jnp.float32),
        jax.ShapeDtypeStruct((nch, 2, _CH), jnp.float32),
    )
    ol_t, oh_t = pl.pallas_call(
        _gates_body,
        in_specs=[
            pl.BlockSpec(memory_space=pltpu.MemorySpace.HBM),
            pl.BlockSpec(memory_space=pltpu.MemorySpace.HBM),
            pl.BlockSpec(memory_space=pltpu.MemorySpace.VMEM),
            pl.BlockSpec(memory_space=pltpu.MemorySpace.VMEM),
            pl.BlockSpec(memory_space=pltpu.MemorySpace.VMEM),
            pl.BlockSpec(memory_space=pltpu.MemorySpace.VMEM),
        ],
        out_specs=(
            pl.BlockSpec(memory_space=pltpu.MemorySpace.VMEM),
            pl.BlockSpec(memory_space=pltpu.MemorySpace.VMEM),
        ),
        out_shape=out_shape,
        scratch_shapes=[
            pltpu.VMEM((_NBUF, _CH, d), jnp.float32),
            pltpu.VMEM((_NBUF, _CH, d), jnp.float32),
            pltpu.SemaphoreType.DMA((2, _NBUF)),
        ],
    )(z_lp, z_hp, w1t, b1r, w2t, b2r)
    x_lp = ol_t.transpose(0, 2, 1).reshape(n, 2)
    x_hp = oh_t.transpose(0, 2, 1).reshape(n, 2)
    return (x_lp, x_hp)
